# baseline (speedup 1.0000x reference)
import functools

import jax
import jax.numpy as jnp
from jax.experimental import pallas as pl

_N_BOXES = 20000
_PRE_TOPK = 6000
_POST_TOPK = 1000
_THRESH = 0.7
_NPAD = 6144
_ROWS = 48
_BLOCKS = 750


def _nms_body(x1r, y1r, x2r, y2r, x1c, y1c, x2c, y2c, keep_ref):
    keep_ref[...] = jnp.ones((_ROWS, 128), jnp.float32)
    lane = jax.lax.broadcasted_iota(jnp.int32, (1, 128), 1)
    gidx = (
        jax.lax.broadcasted_iota(jnp.int32, (_ROWS, 128), 0) * 128
        + jax.lax.broadcasted_iota(jnp.int32, (_ROWS, 128), 1)
    )
    areas = (x2r[...] - x1r[...]) * (y2r[...] - y1r[...])

    def row_body(q, cnt):
        def do_row(c):
            x1q = x1r[pl.ds(q, 1), :]
            y1q = y1r[pl.ds(q, 1), :]
            x2q = x2r[pl.ds(q, 1), :]
            y2q = y2r[pl.ds(q, 1), :]
            areas_q = (x2q - x1q) * (y2q - y1q)
            krow0 = keep_ref[pl.ds(q, 1), :]
            supacc0 = jnp.zeros((_ROWS, 128), jnp.float32)
            n_s = jnp.where(q == 46, 14, 16)

            def sub_body(s, carry):
                krow, supacc = carry
                i = q * 16 + s
                px1 = x1c[pl.ds(i, 1), :]
                py1 = y1c[pl.ds(i, 1), :]
                px2 = x2c[pl.ds(i, 1), :]
                py2 = y2c[pl.ds(i, 1), :]
                for r in range(8):
                    p = i * 8 + r
                    lp = s * 8 + r
                    oh = (lane == lp).astype(jnp.float32)
                    kp = jnp.sum(krow * oh, axis=1, keepdims=True)
                    bx1 = px1[0:1, r : r + 1]
                    by1 = py1[0:1, r : r + 1]
                    bx2 = px2[0:1, r : r + 1]
                    by2 = py2[0:1, r : r + 1]
                    parea = (bx2 - bx1) * (by2 - by1)
                    wq = jnp.clip(jnp.minimum(bx2, x2q) - jnp.maximum(bx1, x1q), 0.0)
                    hq = jnp.clip(jnp.minimum(by2, y2q) - jnp.maximum(by1, y1q), 0.0)
                    interq = wq * hq
                    iouq = interq / (parea + areas_q - interq + 1e-9)
                    supq = ((iouq > _THRESH) & (lane > lp)).astype(jnp.float32)
                    krow = krow * (1.0 - supq * kp)
                    w = jnp.clip(
                        jnp.minimum(bx2, x2r[...]) - jnp.maximum(bx1, x1r[...]), 0.0
                    )
                    h = jnp.clip(
                        jnp.minimum(by2, y2r[...]) - jnp.maximum(by1, y1r[...]), 0.0
                    )
                    inter = w * h
                    iou = inter / (parea + areas - inter + 1e-9)
                    sup = ((iou > _THRESH) & (gidx > p)).astype(jnp.float32)
                    supacc = jnp.maximum(supacc, sup * kp)
                return krow, supacc

            krow, supacc = jax.lax.fori_loop(0, n_s, sub_body, (krow0, supacc0))
            keep_ref[...] = keep_ref[...] * (1.0 - supacc)
            real = ((q * 128 + lane) < _PRE_TOPK).astype(jnp.float32)
            return c + jnp.sum(krow * real)

        return jax.lax.cond(cnt < float(_POST_TOPK), do_row, lambda c: c, cnt)

    jax.lax.fori_loop(0, 47, row_body, 0.0)


@jax.jit
def kernel(boxes, scores):
    top_scores, top_idx = jax.lax.top_k(scores, _PRE_TOPK)
    top_boxes = jnp.take(boxes, top_idx, axis=0)

    pad = _NPAD - _PRE_TOPK
    pb = jnp.pad(top_boxes, ((0, pad), (0, 0)))
    x1 = pb[:, 0]
    y1 = pb[:, 1]
    x2 = pb[:, 2]
    y2 = pb[:, 3]
    row = lambda v: v.reshape(_ROWS, 128)
    col = lambda v: v[:_PRE_TOPK].reshape(_BLOCKS, 8)

    keep = pl.pallas_call(
        _nms_body,
        out_shape=jax.ShapeDtypeStruct((_ROWS, 128), jnp.float32),
    )(row(x1), row(y1), row(x2), row(y2), col(x1), col(y1), col(x2), col(y2))

    keepf = keep.reshape(-1)[:_PRE_TOPK] > 0.5
    masked = jnp.where(keepf, top_scores, -jnp.inf)
    _, sel = jax.lax.top_k(masked, _POST_TOPK)
    final_boxes = jnp.take(top_boxes, sel, axis=0)
    final_scores = jnp.take(top_scores, sel, axis=0)
    return jnp.concatenate([final_boxes, final_scores[:, None]], axis=1)


# speedup vs baseline: 1.0302x; 1.0302x over previous
import functools

import jax
import jax.numpy as jnp
from jax.experimental import pallas as pl

_N_BOXES = 20000
_PRE_TOPK = 6000
_POST_TOPK = 1000
_THRESH = 0.7
_NPAD = 6144
_ROWS = 48
_BLOCKS = 750


def _nms_body(x1r, y1r, x2r, y2r, x1c, y1c, x2c, y2c, keep_ref):
    keep_ref[...] = jnp.ones((_ROWS, 128), jnp.float32)
    lane = jax.lax.broadcasted_iota(jnp.int32, (1, 128), 1)
    gidx = (
        jax.lax.broadcasted_iota(jnp.int32, (_ROWS, 128), 0) * 128
        + jax.lax.broadcasted_iota(jnp.int32, (_ROWS, 128), 1)
    )
    areas = (x2r[...] - x1r[...]) * (y2r[...] - y1r[...])

    def row_body(q, cnt):
        def do_row(c):
            x1q = x1r[pl.ds(q, 1), :]
            y1q = y1r[pl.ds(q, 1), :]
            x2q = x2r[pl.ds(q, 1), :]
            y2q = y2r[pl.ds(q, 1), :]
            areas_q = (x2q - x1q) * (y2q - y1q)
            krow0 = keep_ref[pl.ds(q, 1), :]
            supacc0 = jnp.zeros((_ROWS, 128), jnp.float32)
            n_s = jnp.where(q == 46, 7, 8)

            def sub_body(s, carry):
                krow, supacc = carry
                i = q * 8 + s
                px1 = jnp.reshape(x1c[pl.ds(i, 1)], (16, 1))
                py1 = jnp.reshape(y1c[pl.ds(i, 1)], (16, 1))
                px2 = jnp.reshape(x2c[pl.ds(i, 1)], (16, 1))
                py2 = jnp.reshape(y2c[pl.ds(i, 1)], (16, 1))
                for r in range(16):
                    p = i * 16 + r
                    lp = s * 16 + r
                    oh = (lane == lp).astype(jnp.float32)
                    kp = jnp.sum(krow * oh, axis=1, keepdims=True)
                    bx1 = px1[r : r + 1, :]
                    by1 = py1[r : r + 1, :]
                    bx2 = px2[r : r + 1, :]
                    by2 = py2[r : r + 1, :]
                    parea = (bx2 - bx1) * (by2 - by1)
                    wq = jnp.clip(jnp.minimum(bx2, x2q) - jnp.maximum(bx1, x1q), 0.0)
                    hq = jnp.clip(jnp.minimum(by2, y2q) - jnp.maximum(by1, y1q), 0.0)
                    interq = wq * hq
                    iouq = interq / (parea + areas_q - interq + 1e-9)
                    supq = ((iouq > _THRESH) & (lane > lp)).astype(jnp.float32)
                    krow = krow * (1.0 - supq * kp)
                    w = jnp.clip(
                        jnp.minimum(bx2, x2r[...]) - jnp.maximum(bx1, x1r[...]), 0.0
                    )
                    h = jnp.clip(
                        jnp.minimum(by2, y2r[...]) - jnp.maximum(by1, y1r[...]), 0.0
                    )
                    inter = w * h
                    iou = inter / (parea + areas - inter + 1e-9)
                    sup = ((iou > _THRESH) & (gidx > p)).astype(jnp.float32)
                    supacc = jnp.maximum(supacc, sup * kp)
                return krow, supacc

            krow, supacc = jax.lax.fori_loop(0, n_s, sub_body, (krow0, supacc0))
            keep_ref[...] = keep_ref[...] * (1.0 - supacc)
            real = ((q * 128 + lane) < _PRE_TOPK).astype(jnp.float32)
            return c + jnp.sum(krow * real)

        return jax.lax.cond(cnt < float(_POST_TOPK), do_row, lambda c: c, cnt)

    jax.lax.fori_loop(0, 47, row_body, 0.0)


@jax.jit
def kernel(boxes, scores):
    top_scores, top_idx = jax.lax.top_k(scores, _PRE_TOPK)
    top_boxes = jnp.take(boxes, top_idx, axis=0)

    pad = _NPAD - _PRE_TOPK
    pb = jnp.pad(top_boxes, ((0, pad), (0, 0)))
    x1 = pb[:, 0]
    y1 = pb[:, 1]
    x2 = pb[:, 2]
    y2 = pb[:, 3]
    row = lambda v: v.reshape(_ROWS, 128)
    col = lambda v: v[:_PRE_TOPK].reshape(375, 16, 1)

    keep = pl.pallas_call(
        _nms_body,
        out_shape=jax.ShapeDtypeStruct((_ROWS, 128), jnp.float32),
    )(row(x1), row(y1), row(x2), row(y2), col(x1), col(y1), col(x2), col(y2))

    keepf = keep.reshape(-1)[:_PRE_TOPK] > 0.5
    masked = jnp.where(keepf, top_scores, -jnp.inf)
    _, sel = jax.lax.top_k(masked, _POST_TOPK)
    final_boxes = jnp.take(top_boxes, sel, axis=0)
    final_scores = jnp.take(top_scores, sel, axis=0)
    return jnp.concatenate([final_boxes, final_scores[:, None]], axis=1)
